# baseline (speedup 1.0000x reference)
import functools

import jax
import jax.numpy as jnp
from jax import lax
from jax.experimental import pallas as pl
from jax.experimental.pallas import tpu as pltpu
from jax.experimental.pallas import tpu_sc as plsc

N_NODES = 10000
D = 128
NC = 2
NS = 16
NW = NC * NS
C = 128
ZB = 400
NZB = N_NODES // ZB


def _sc_mesh():
    return plsc.VectorSubcoreMesh(core_axis_name="c", subcore_axis_name="s")


def _run_pipelined_gather(wid, table_h, sidx_h, ridx_h, sout_h, rout_h,
                          idxbufs, rowbufs, sems, A, CH, K):
    si, ri = idxbufs[0:2], idxbufs[2:4]
    sr, rr = rowbufs[0:2], rowbufs[2:4]
    sis, ris = sems[0:2], sems[2:4]
    gss, grs = sems[4:6], sems[6:8]
    wss, wrs = sems[8:10], sems[10:12]

    @pl.when(wid < A)
    def _():
        tb = wid * (CH * K)

        def fire_idx(k, p):
            base = tb + k * CH
            pltpu.async_copy(sidx_h.at[pl.ds(base, CH)], si[p], sis[p])
            pltpu.async_copy(ridx_h.at[pl.ds(base, CH)], ri[p], ris[p])

        def step(k, p):
            base = tb + k * CH
            pltpu.make_async_copy(
                sidx_h.at[pl.ds(base, CH)], si[p], sis[p]).wait()
            pltpu.make_async_copy(
                ridx_h.at[pl.ds(base, CH)], ri[p], ris[p]).wait()

            @pl.when(k >= 2)
            def _():
                pltpu.make_async_copy(
                    sr[p], sout_h.at[pl.ds(base, CH)], wss[p]).wait()
                pltpu.make_async_copy(
                    rr[p], rout_h.at[pl.ds(base, CH)], wrs[p]).wait()

            cp_s = pltpu.async_copy(table_h.at[si[p]], sr[p], gss[p])
            cp_r = pltpu.async_copy(table_h.at[ri[p]], rr[p], grs[p])
            cp_s.wait()
            cp_r.wait()

            @pl.when(k + 2 < K)
            def _():
                fire_idx(k + 2, p)

            pltpu.async_copy(sr[p], sout_h.at[pl.ds(base, CH)], wss[p])
            pltpu.async_copy(rr[p], rout_h.at[pl.ds(base, CH)], wrs[p])

        fire_idx(0, 0)
        fire_idx(1, 1)

        def pair(i, carry):
            step(2 * i, 0)
            step(2 * i + 1, 1)
            return carry

        lax.fori_loop(0, K // 2, pair, 0)
        if K % 2:
            step(K - 1, (K - 1) % 2)
        for p in range(2):
            pltpu.make_async_copy(
                sr[p], sout_h.at[pl.ds(tb, CH)], wss[p]).wait()
            pltpu.make_async_copy(
                rr[p], rout_h.at[pl.ds(tb, CH)], wrs[p]).wait()


def _gather_all(table, m_s, m_r, w_s, w_r):
    EM, EW = m_s.shape[0], w_s.shape[0]
    MA, MC, MK = 32, 80, EM // (32 * 80)
    WA, WC, WK = 25, 80, EW // (25 * 80)
    assert MA * MC * MK == EM and WA * WC * WK == EW

    @functools.partial(
        pl.kernel,
        out_type=[jax.ShapeDtypeStruct((EM, D), jnp.float32)] * 2
        + [jax.ShapeDtypeStruct((EW, D), jnp.float32)] * 2,
        mesh=_sc_mesh(),
        scratch_types=(
            [pltpu.VMEM((MC,), jnp.int32)] * 4
            + [pltpu.VMEM((MC, D), jnp.float32)] * 4
            + [pltpu.SemaphoreType.DMA] * 24
        ),
    )
    def gather_kernel(table_h, msi_h, mri_h, wsi_h, wri_h,
                      mso_h, mro_h, wso_h, wro_h, *scr):
        wid = lax.axis_index("s") * NC + lax.axis_index("c")
        _run_pipelined_gather(wid, table_h, msi_h, mri_h, mso_h, mro_h,
                              scr[0:4], scr[4:8], scr[8:20], MA, MC, MK)
        _run_pipelined_gather(wid, table_h, wsi_h, wri_h, wso_h, wro_h,
                              scr[0:4], scr[4:8], scr[20:32], WA, WC, WK)

    return gather_kernel(table, m_s, m_r, w_s, w_r)


def _run_pipelined_scatter(wid, vals_h, ridx_h, acc_sh, bufs, sems, A, CH, K):
    idx, val = bufs[0:2], bufs[2:4]
    isem, vsem, ssem = sems[0:2], sems[2:4], sems[4:6]

    @pl.when(wid < A)
    def _():
        tb = wid * (CH * K)

        def fire_iv(k, p):
            base = tb + k * CH
            pltpu.async_copy(ridx_h.at[pl.ds(base, CH)], idx[p], isem[p])
            pltpu.async_copy(vals_h.at[pl.ds(base, CH)], val[p], vsem[p])

        def step(k, p):
            base = tb + k * CH
            pltpu.make_async_copy(
                ridx_h.at[pl.ds(base, CH)], idx[p], isem[p]).wait()
            pltpu.make_async_copy(
                vals_h.at[pl.ds(base, CH)], val[p], vsem[p]).wait()
            cp = pltpu.async_copy(val[p], acc_sh.at[idx[p]], ssem[p],
                                  add=True)
            cp.wait()

            @pl.when(k + 2 < K)
            def _():
                fire_iv(k + 2, p)

        fire_iv(0, 0)
        fire_iv(1, 1)

        def pair(i, carry):
            step(2 * i, 0)
            step(2 * i + 1, 1)
            return carry

        lax.fori_loop(0, K // 2, pair, 0)
        if K % 2:
            step(K - 1, (K - 1) % 2)


def _segment_sums(mlp_m, m_ridx, mlp_w, w_ridx, zeros_blk):
    EM, EW = mlp_m.shape[0], mlp_w.shape[0]
    MA, MC, MK = 32, 80, EM // (32 * 80)
    WA, WC, WK = 25, 80, EW // (25 * 80)
    assert MA * MC * MK == EM and WA * WC * WK == EW

    @functools.partial(
        pl.kernel,
        out_type=[jax.ShapeDtypeStruct((NC, N_NODES, D), jnp.float32)] * 2,
        mesh=_sc_mesh(),
        scratch_types=(
            [pltpu.VMEM((MC,), jnp.int32)] * 2
            + [pltpu.VMEM((MC, D), jnp.float32)] * 2
            + [pltpu.VMEM_SHARED((N_NODES, D), jnp.float32)]
            + [pltpu.SemaphoreType.DMA] * 12
        ),
    )
    def scatter_kernel(mvals_h, mridx_h, wvals_h, wridx_h, zeros_h,
                       mout_h, wout_h, *scr):
        acc_sh = scr[4]
        cid = lax.axis_index("c")
        sid = lax.axis_index("s")
        wid = sid * NC + cid

        def zero_acc():
            for kz in range(-(-NZB // NS)):
                b = sid + kz * NS

                @pl.when(b < NZB)
                def _():
                    pltpu.sync_copy(zeros_h, acc_sh.at[pl.ds(b * ZB, ZB)])

        def dump(out_h):
            for kd in range(-(-NZB // NS)):
                b = sid + kd * NS

                @pl.when(b < NZB)
                def _():
                    pltpu.sync_copy(acc_sh.at[pl.ds(b * ZB, ZB)],
                                    out_h.at[cid, pl.ds(b * ZB, ZB)])

        zero_acc()
        plsc.subcore_barrier()
        _run_pipelined_scatter(wid, mvals_h, mridx_h, acc_sh,
                               scr[0:4], scr[5:11], MA, MC, MK)
        plsc.subcore_barrier()
        dump(mout_h)
        plsc.subcore_barrier()
        zero_acc()
        plsc.subcore_barrier()
        _run_pipelined_scatter(wid, wvals_h, wridx_h, acc_sh,
                               scr[0:4], scr[11:17], WA, WC, WK)
        plsc.subcore_barrier()
        dump(wout_h)

    return scatter_kernel(mlp_m, m_ridx, mlp_w, w_ridx, zeros_blk)


def _edge_mlp(gs, gr, ef, W1, b1, W2, b2, block):
    E = gs.shape[0]

    def body(gs_r, gr_r, ef_r, w1_r, b1_r, w2_r, b2_r, mlp_r, new_r):
        w1 = w1_r[...].astype(jnp.bfloat16)
        ef_blk = ef_r[...]
        x = (jnp.dot(gs_r[...].astype(jnp.bfloat16), w1[0:D],
                     preferred_element_type=jnp.float32)
             + jnp.dot(gr_r[...].astype(jnp.bfloat16), w1[D:2 * D],
                       preferred_element_type=jnp.float32)
             + jnp.dot(ef_blk.astype(jnp.bfloat16), w1[2 * D:3 * D],
                       preferred_element_type=jnp.float32)
             + b1_r[...])
        h = jnp.maximum(x, 0.0).astype(jnp.bfloat16)
        y = (jnp.dot(h, w2_r[...].astype(jnp.bfloat16),
                     preferred_element_type=jnp.float32) + b2_r[...])
        mlp_r[...] = y
        new_r[...] = y + ef_blk

    full = lambda i: (0, 0)
    blk = lambda i: (i, 0)
    return pl.pallas_call(
        body,
        grid=(E // block,),
        in_specs=[
            pl.BlockSpec((block, D), blk),
            pl.BlockSpec((block, D), blk),
            pl.BlockSpec((block, D), blk),
            pl.BlockSpec((3 * D, D), full),
            pl.BlockSpec((1, D), full),
            pl.BlockSpec((D, D), full),
            pl.BlockSpec((1, D), full),
        ],
        out_specs=[pl.BlockSpec((block, D), blk)] * 2,
        out_shape=[jax.ShapeDtypeStruct((E, D), jnp.float32)] * 2,
    )(gs, gr, ef, W1, b1.reshape(1, D), W2, b2.reshape(1, D))


def _node_mlp(nodes, aggm, aggw, W1, b1, W2, b2, block):
    def body(n_r, am_r, aw_r, w1_r, b1_r, w2_r, b2_r, out_r):
        w1 = w1_r[...].astype(jnp.bfloat16)
        nf = n_r[...]
        am = (am_r[0] + am_r[1]).astype(jnp.bfloat16)
        aw = (aw_r[0] + aw_r[1]).astype(jnp.bfloat16)
        x = (jnp.dot(nf.astype(jnp.bfloat16), w1[0:D],
                     preferred_element_type=jnp.float32)
             + jnp.dot(am, w1[D:2 * D], preferred_element_type=jnp.float32)
             + jnp.dot(aw, w1[2 * D:3 * D], preferred_element_type=jnp.float32)
             + b1_r[...])
        h = jnp.maximum(x, 0.0).astype(jnp.bfloat16)
        out_r[...] = (jnp.dot(h, w2_r[...].astype(jnp.bfloat16),
                              preferred_element_type=jnp.float32)
                      + b2_r[...] + nf)

    full = lambda i: (0, 0)
    blk = lambda i: (i, 0)
    pblk = lambda i: (0, i, 0)
    return pl.pallas_call(
        body,
        grid=(N_NODES // block,),
        in_specs=[
            pl.BlockSpec((block, D), blk),
            pl.BlockSpec((NC, block, D), pblk),
            pl.BlockSpec((NC, block, D), pblk),
            pl.BlockSpec((3 * D, D), full),
            pl.BlockSpec((1, D), full),
            pl.BlockSpec((D, D), full),
            pl.BlockSpec((1, D), full),
        ],
        out_specs=pl.BlockSpec((block, D), blk),
        out_shape=jax.ShapeDtypeStruct((N_NODES, D), jnp.float32),
    )(nodes, aggm, aggw, W1, b1.reshape(1, D), W2, b2.reshape(1, D))


def kernel(node_features, mesh_edge_features, world_edge_features,
           mesh_senders, mesh_receivers, world_senders, world_receivers,
           mesh_W1, mesh_b1, mesh_W2, mesh_b2,
           world_W1, world_b1, world_W2, world_b2,
           node_W1, node_b1, node_W2, node_b2):
    ms_rows, mr_rows, ws_rows, wr_rows = _gather_all(
        node_features, mesh_senders, mesh_receivers,
        world_senders, world_receivers)

    mlp_m, new_mesh = _edge_mlp(ms_rows, mr_rows, mesh_edge_features,
                                mesh_W1, mesh_b1, mesh_W2, mesh_b2, 1000)
    mlp_w, new_world = _edge_mlp(ws_rows, wr_rows, world_edge_features,
                                 world_W1, world_b1, world_W2, world_b2, 1000)

    zeros_blk = jnp.zeros((ZB, D), jnp.float32)
    aggm, aggw = _segment_sums(mlp_m, mesh_receivers,
                               mlp_w, world_receivers, zeros_blk)

    new_node = _node_mlp(node_features, aggm, aggw,
                         node_W1, node_b1, node_W2, node_b2, 1000)
    return (new_node, new_mesh, new_world)


# speedup vs baseline: 1.1258x; 1.1258x over previous
import functools

import jax
import jax.numpy as jnp
from jax import lax
from jax.experimental import pallas as pl
from jax.experimental.pallas import tpu as pltpu
from jax.experimental.pallas import tpu_sc as plsc

N_NODES = 10000
D = 128
NC = 2
NS = 16
NW = NC * NS
C = 128
ZB = 400
NZB = N_NODES // ZB


def _sc_mesh():
    return plsc.VectorSubcoreMesh(core_axis_name="c", subcore_axis_name="s")


def _run_pipelined_gather(wid, table_h, sidx_h, ridx_h, sout_h, rout_h,
                          idxbufs, rowbufs, sems, A, CH, K):
    si, ri = idxbufs[0:2], idxbufs[2:4]
    sr, rr = rowbufs[0:2], rowbufs[2:4]
    sis, ris = sems[0:2], sems[2:4]
    gss, grs = sems[4:6], sems[6:8]
    wss, wrs = sems[8:10], sems[10:12]

    @pl.when(wid < A)
    def _():
        tb = wid * (CH * K)

        def fire_idx(k, p):
            base = tb + k * CH
            pltpu.async_copy(sidx_h.at[pl.ds(base, CH)], si[p], sis[p])
            pltpu.async_copy(ridx_h.at[pl.ds(base, CH)], ri[p], ris[p])

        def step(k, p):
            base = tb + k * CH
            pltpu.make_async_copy(
                sidx_h.at[pl.ds(base, CH)], si[p], sis[p]).wait()
            pltpu.make_async_copy(
                ridx_h.at[pl.ds(base, CH)], ri[p], ris[p]).wait()

            @pl.when(k >= 2)
            def _():
                pltpu.make_async_copy(
                    sr[p], sout_h.at[pl.ds(base, CH)], wss[p]).wait()
                pltpu.make_async_copy(
                    rr[p], rout_h.at[pl.ds(base, CH)], wrs[p]).wait()

            cp_s = pltpu.async_copy(table_h.at[si[p]], sr[p], gss[p])
            cp_r = pltpu.async_copy(table_h.at[ri[p]], rr[p], grs[p])
            cp_s.wait()
            cp_r.wait()

            @pl.when(k + 2 < K)
            def _():
                fire_idx(k + 2, p)

            pltpu.async_copy(sr[p], sout_h.at[pl.ds(base, CH)], wss[p])
            pltpu.async_copy(rr[p], rout_h.at[pl.ds(base, CH)], wrs[p])

        fire_idx(0, 0)
        fire_idx(1, 1)

        def pair(i, carry):
            step(2 * i, 0)
            step(2 * i + 1, 1)
            return carry

        lax.fori_loop(0, K // 2, pair, 0)
        if K % 2:
            step(K - 1, (K - 1) % 2)
        for p in range(2):
            pltpu.make_async_copy(
                sr[p], sout_h.at[pl.ds(tb, CH)], wss[p]).wait()
            pltpu.make_async_copy(
                rr[p], rout_h.at[pl.ds(tb, CH)], wrs[p]).wait()


def _gather_pairs(table, sidx, ridx, A, CH):
    E = sidx.shape[0]
    K = E // (A * CH)
    assert A * CH * K == E and K >= 2

    @functools.partial(
        pl.kernel,
        out_type=[jax.ShapeDtypeStruct((E, D), jnp.float32)] * 2,
        mesh=_sc_mesh(),
        scratch_types=(
            [pltpu.VMEM((CH,), jnp.int32)] * 4
            + [pltpu.VMEM((CH, D), jnp.float32)] * 4
            + [pltpu.SemaphoreType.DMA] * 12
        ),
    )
    def gather_kernel(table_h, sidx_h, ridx_h, sout_h, rout_h, *scr):
        wid = lax.axis_index("s") * NC + lax.axis_index("c")
        _run_pipelined_gather(wid, table_h, sidx_h, ridx_h, sout_h, rout_h,
                              scr[0:4], scr[4:8], scr[8:20], A, CH, K)

    return gather_kernel(table, sidx, ridx)


def _run_pipelined_scatter(wid, vals_h, ridx_h, acc_sh, bufs, sems, A, CH, K):
    idx, val = bufs[0:2], bufs[2:4]
    isem, vsem, ssem = sems[0:2], sems[2:4], sems[4:6]

    @pl.when(wid < A)
    def _():
        tb = wid * (CH * K)

        def fire_iv(k, p):
            base = tb + k * CH
            pltpu.async_copy(ridx_h.at[pl.ds(base, CH)], idx[p], isem[p])
            pltpu.async_copy(vals_h.at[pl.ds(base, CH)], val[p], vsem[p])

        def step(k, p):
            base = tb + k * CH
            pltpu.make_async_copy(
                ridx_h.at[pl.ds(base, CH)], idx[p], isem[p]).wait()
            pltpu.make_async_copy(
                vals_h.at[pl.ds(base, CH)], val[p], vsem[p]).wait()
            cp = pltpu.async_copy(val[p], acc_sh.at[idx[p]], ssem[p],
                                  add=True)
            cp.wait()

            @pl.when(k + 2 < K)
            def _():
                fire_iv(k + 2, p)

        fire_iv(0, 0)
        fire_iv(1, 1)

        def pair(i, carry):
            step(2 * i, 0)
            step(2 * i + 1, 1)
            return carry

        lax.fori_loop(0, K // 2, pair, 0)
        if K % 2:
            step(K - 1, (K - 1) % 2)


def _segment_sum_partials(vals, ridx, zeros_blk, A, CH):
    E = vals.shape[0]
    K = E // (A * CH)
    assert A * CH * K == E and K >= 2

    @functools.partial(
        pl.kernel,
        out_type=jax.ShapeDtypeStruct((NC, N_NODES, D), jnp.float32),
        mesh=_sc_mesh(),
        scratch_types=(
            [pltpu.VMEM((CH,), jnp.int32)] * 2
            + [pltpu.VMEM((CH, D), jnp.float32)] * 2
            + [pltpu.VMEM_SHARED((N_NODES, D), jnp.float32)]
            + [pltpu.SemaphoreType.DMA] * 6
        ),
    )
    def scatter_kernel(vals_h, ridx_h, zeros_h, out_h, *scr):
        acc_sh = scr[4]
        cid = lax.axis_index("c")
        sid = lax.axis_index("s")
        wid = sid * NC + cid

        for kz in range(-(-NZB // NS)):
            b = sid + kz * NS

            @pl.when(b < NZB)
            def _():
                pltpu.sync_copy(zeros_h, acc_sh.at[pl.ds(b * ZB, ZB)])

        plsc.subcore_barrier()
        _run_pipelined_scatter(wid, vals_h, ridx_h, acc_sh,
                               scr[0:4], scr[5:11], A, CH, K)
        plsc.subcore_barrier()

        for kd in range(-(-NZB // NS)):
            b = sid + kd * NS

            @pl.when(b < NZB)
            def _():
                pltpu.sync_copy(acc_sh.at[pl.ds(b * ZB, ZB)],
                                out_h.at[cid, pl.ds(b * ZB, ZB)])

    return scatter_kernel(vals, ridx, zeros_blk)


def _edge_mlp(gs, gr, ef, W1, b1, W2, b2, block):
    E = gs.shape[0]

    def body(gs_r, gr_r, ef_r, w1_r, b1_r, w2_r, b2_r, mlp_r, new_r):
        w1 = w1_r[...].astype(jnp.bfloat16)
        ef_blk = ef_r[...]
        x = (jnp.dot(gs_r[...].astype(jnp.bfloat16), w1[0:D],
                     preferred_element_type=jnp.float32)
             + jnp.dot(gr_r[...].astype(jnp.bfloat16), w1[D:2 * D],
                       preferred_element_type=jnp.float32)
             + jnp.dot(ef_blk.astype(jnp.bfloat16), w1[2 * D:3 * D],
                       preferred_element_type=jnp.float32)
             + b1_r[...])
        h = jnp.maximum(x, 0.0).astype(jnp.bfloat16)
        y = (jnp.dot(h, w2_r[...].astype(jnp.bfloat16),
                     preferred_element_type=jnp.float32) + b2_r[...])
        mlp_r[...] = y
        new_r[...] = y + ef_blk

    full = lambda i: (0, 0)
    blk = lambda i: (i, 0)
    return pl.pallas_call(
        body,
        grid=(E // block,),
        in_specs=[
            pl.BlockSpec((block, D), blk),
            pl.BlockSpec((block, D), blk),
            pl.BlockSpec((block, D), blk),
            pl.BlockSpec((3 * D, D), full),
            pl.BlockSpec((1, D), full),
            pl.BlockSpec((D, D), full),
            pl.BlockSpec((1, D), full),
        ],
        out_specs=[pl.BlockSpec((block, D), blk)] * 2,
        out_shape=[jax.ShapeDtypeStruct((E, D), jnp.float32)] * 2,
    )(gs, gr, ef, W1, b1.reshape(1, D), W2, b2.reshape(1, D))


def _node_mlp(nodes, aggm, aggw, W1, b1, W2, b2, block):
    def body(n_r, am_r, aw_r, w1_r, b1_r, w2_r, b2_r, out_r):
        w1 = w1_r[...].astype(jnp.bfloat16)
        nf = n_r[...]
        am = (am_r[0] + am_r[1]).astype(jnp.bfloat16)
        aw = (aw_r[0] + aw_r[1]).astype(jnp.bfloat16)
        x = (jnp.dot(nf.astype(jnp.bfloat16), w1[0:D],
                     preferred_element_type=jnp.float32)
             + jnp.dot(am, w1[D:2 * D], preferred_element_type=jnp.float32)
             + jnp.dot(aw, w1[2 * D:3 * D], preferred_element_type=jnp.float32)
             + b1_r[...])
        h = jnp.maximum(x, 0.0).astype(jnp.bfloat16)
        out_r[...] = (jnp.dot(h, w2_r[...].astype(jnp.bfloat16),
                              preferred_element_type=jnp.float32)
                      + b2_r[...] + nf)

    full = lambda i: (0, 0)
    blk = lambda i: (i, 0)
    pblk = lambda i: (0, i, 0)
    return pl.pallas_call(
        body,
        grid=(N_NODES // block,),
        in_specs=[
            pl.BlockSpec((block, D), blk),
            pl.BlockSpec((NC, block, D), pblk),
            pl.BlockSpec((NC, block, D), pblk),
            pl.BlockSpec((3 * D, D), full),
            pl.BlockSpec((1, D), full),
            pl.BlockSpec((D, D), full),
            pl.BlockSpec((1, D), full),
        ],
        out_specs=pl.BlockSpec((block, D), blk),
        out_shape=jax.ShapeDtypeStruct((N_NODES, D), jnp.float32),
    )(nodes, aggm, aggw, W1, b1.reshape(1, D), W2, b2.reshape(1, D))


def kernel(node_features, mesh_edge_features, world_edge_features,
           mesh_senders, mesh_receivers, world_senders, world_receivers,
           mesh_W1, mesh_b1, mesh_W2, mesh_b2,
           world_W1, world_b1, world_W2, world_b2,
           node_W1, node_b1, node_W2, node_b2):
    ms_rows, mr_rows = _gather_pairs(node_features, mesh_senders,
                                     mesh_receivers, 32, 80)
    ws_rows, wr_rows = _gather_pairs(node_features, world_senders,
                                     world_receivers, 25, 128)

    mlp_m, new_mesh = _edge_mlp(ms_rows, mr_rows, mesh_edge_features,
                                mesh_W1, mesh_b1, mesh_W2, mesh_b2, 1000)
    mlp_w, new_world = _edge_mlp(ws_rows, wr_rows, world_edge_features,
                                 world_W1, world_b1, world_W2, world_b2, 1000)

    zeros_blk = jnp.zeros((ZB, D), jnp.float32)
    aggm = _segment_sum_partials(mlp_m, mesh_receivers, zeros_blk, 32, 80)
    aggw = _segment_sum_partials(mlp_w, world_receivers, zeros_blk, 25, 128)

    new_node = _node_mlp(node_features, aggm, aggw,
                         node_W1, node_b1, node_W2, node_b2, 1000)
    return (new_node, new_mesh, new_world)


# speedup vs baseline: 1.1263x; 1.0005x over previous
import functools

import jax
import jax.numpy as jnp
from jax import lax
from jax.experimental import pallas as pl
from jax.experimental.pallas import tpu as pltpu
from jax.experimental.pallas import tpu_sc as plsc

N_NODES = 10000
D = 128
NC = 2
NS = 16
NW = NC * NS
C = 128
ZB = 400
NZB = N_NODES // ZB


def _sc_mesh():
    return plsc.VectorSubcoreMesh(core_axis_name="c", subcore_axis_name="s")


def _run_pipelined_gather(wid, table_h, sidx_h, ridx_h, sout_h, rout_h,
                          idxbufs, rowbufs, sems, A, CH, K):
    si, ri = idxbufs[0:2], idxbufs[2:4]
    sr, rr = rowbufs[0:2], rowbufs[2:4]
    sis, ris = sems[0:2], sems[2:4]
    gss, grs = sems[4:6], sems[6:8]
    wss, wrs = sems[8:10], sems[10:12]

    @pl.when(wid < A)
    def _():
        tb = wid * (CH * K)

        def fire_idx(k, p):
            base = tb + k * CH
            pltpu.async_copy(sidx_h.at[pl.ds(base, CH)], si[p], sis[p])
            pltpu.async_copy(ridx_h.at[pl.ds(base, CH)], ri[p], ris[p])

        def step(k, p):
            base = tb + k * CH
            pltpu.make_async_copy(
                sidx_h.at[pl.ds(base, CH)], si[p], sis[p]).wait()
            pltpu.make_async_copy(
                ridx_h.at[pl.ds(base, CH)], ri[p], ris[p]).wait()

            @pl.when(k >= 2)
            def _():
                pltpu.make_async_copy(
                    sr[p], sout_h.at[pl.ds(base, CH)], wss[p]).wait()
                pltpu.make_async_copy(
                    rr[p], rout_h.at[pl.ds(base, CH)], wrs[p]).wait()

            cp_s = pltpu.async_copy(table_h.at[si[p]], sr[p], gss[p])
            cp_r = pltpu.async_copy(table_h.at[ri[p]], rr[p], grs[p])
            cp_s.wait()
            cp_r.wait()

            @pl.when(k + 2 < K)
            def _():
                fire_idx(k + 2, p)

            pltpu.async_copy(sr[p], sout_h.at[pl.ds(base, CH)], wss[p])
            pltpu.async_copy(rr[p], rout_h.at[pl.ds(base, CH)], wrs[p])

        fire_idx(0, 0)
        fire_idx(1, 1)

        def pair(i, carry):
            step(2 * i, 0)
            step(2 * i + 1, 1)
            return carry

        lax.fori_loop(0, K // 2, pair, 0)
        if K % 2:
            step(K - 1, (K - 1) % 2)
        for p in range(2):
            pltpu.make_async_copy(
                sr[p], sout_h.at[pl.ds(tb, CH)], wss[p]).wait()
            pltpu.make_async_copy(
                rr[p], rout_h.at[pl.ds(tb, CH)], wrs[p]).wait()


def _gather_pairs(table, sidx, ridx, A, CH):
    E = sidx.shape[0]
    K = E // (A * CH)
    assert A * CH * K == E and K >= 2

    @functools.partial(
        pl.kernel,
        out_type=[jax.ShapeDtypeStruct((E, D), jnp.float32)] * 2,
        mesh=_sc_mesh(),
        scratch_types=(
            [pltpu.VMEM((CH,), jnp.int32)] * 4
            + [pltpu.VMEM((CH, D), jnp.float32)] * 4
            + [pltpu.SemaphoreType.DMA] * 12
        ),
    )
    def gather_kernel(table_h, sidx_h, ridx_h, sout_h, rout_h, *scr):
        wid = lax.axis_index("s") * NC + lax.axis_index("c")
        _run_pipelined_gather(wid, table_h, sidx_h, ridx_h, sout_h, rout_h,
                              scr[0:4], scr[4:8], scr[8:20], A, CH, K)

    return gather_kernel(table, sidx, ridx)


def _run_pipelined_scatter(wid, vals_h, ridx_h, acc_sh, bufs, sems, A, CH, K):
    idx, val = bufs[0:2], bufs[2:4]
    isem, vsem, ssem = sems[0:2], sems[2:4], sems[4:6]

    @pl.when(wid < A)
    def _():
        tb = wid * (CH * K)

        def fire_iv(k, p):
            base = tb + k * CH
            pltpu.async_copy(ridx_h.at[pl.ds(base, CH)], idx[p], isem[p])
            pltpu.async_copy(vals_h.at[pl.ds(base, CH)], val[p], vsem[p])

        def step(k, p):
            base = tb + k * CH
            pltpu.make_async_copy(
                ridx_h.at[pl.ds(base, CH)], idx[p], isem[p]).wait()
            pltpu.make_async_copy(
                vals_h.at[pl.ds(base, CH)], val[p], vsem[p]).wait()
            cp = pltpu.async_copy(val[p], acc_sh.at[idx[p]], ssem[p],
                                  add=True)
            cp.wait()

            @pl.when(k + 2 < K)
            def _():
                fire_iv(k + 2, p)

        fire_iv(0, 0)
        fire_iv(1, 1)

        def pair(i, carry):
            step(2 * i, 0)
            step(2 * i + 1, 1)
            return carry

        lax.fori_loop(0, K // 2, pair, 0)
        if K % 2:
            step(K - 1, (K - 1) % 2)


def _segment_sum_partials(vals, ridx, zeros_blk, A, CH):
    E = vals.shape[0]
    K = E // (A * CH)
    assert A * CH * K == E and K >= 2

    @functools.partial(
        pl.kernel,
        out_type=jax.ShapeDtypeStruct((NC, N_NODES, D), jnp.float32),
        mesh=_sc_mesh(),
        scratch_types=(
            [pltpu.VMEM((CH,), jnp.int32)] * 2
            + [pltpu.VMEM((CH, D), jnp.float32)] * 2
            + [pltpu.VMEM_SHARED((N_NODES, D), jnp.float32)]
            + [pltpu.SMEM((1,), jnp.int32)]
            + [pltpu.SemaphoreType.DMA] * 6
        ),
    )
    def scatter_kernel(vals_h, ridx_h, zeros_h, out_h, *scr):
        acc_sh = scr[4]
        spin_ref = scr[5]
        cid = lax.axis_index("c")
        sid = lax.axis_index("s")
        wid = sid * NC + cid

        def settle(n):
            spin_ref[0] = lax.fori_loop(0, n, lambda i, c: c + 1,
                                        spin_ref[0])

        for kz in range(-(-NZB // NS)):
            b = sid + kz * NS

            @pl.when(b < NZB)
            def _():
                pltpu.sync_copy(zeros_h, acc_sh.at[pl.ds(b * ZB, ZB)])

        settle(1024)
        plsc.subcore_barrier()
        _run_pipelined_scatter(wid, vals_h, ridx_h, acc_sh,
                               scr[0:4], scr[6:12], A, CH, K)
        settle(1024)
        plsc.subcore_barrier()

        for kd in range(-(-NZB // NS)):
            b = sid + kd * NS

            @pl.when(b < NZB)
            def _():
                pltpu.sync_copy(acc_sh.at[pl.ds(b * ZB, ZB)],
                                out_h.at[cid, pl.ds(b * ZB, ZB)])

    return scatter_kernel(vals, ridx, zeros_blk)


def _edge_mlp(gs, gr, ef, W1, b1, W2, b2, block):
    E = gs.shape[0]

    def body(gs_r, gr_r, ef_r, w1_r, b1_r, w2_r, b2_r, mlp_r, new_r):
        w1 = w1_r[...].astype(jnp.bfloat16)
        ef_blk = ef_r[...]
        x = (jnp.dot(gs_r[...].astype(jnp.bfloat16), w1[0:D],
                     preferred_element_type=jnp.float32)
             + jnp.dot(gr_r[...].astype(jnp.bfloat16), w1[D:2 * D],
                       preferred_element_type=jnp.float32)
             + jnp.dot(ef_blk.astype(jnp.bfloat16), w1[2 * D:3 * D],
                       preferred_element_type=jnp.float32)
             + b1_r[...])
        h = jnp.maximum(x, 0.0).astype(jnp.bfloat16)
        y = (jnp.dot(h, w2_r[...].astype(jnp.bfloat16),
                     preferred_element_type=jnp.float32) + b2_r[...])
        mlp_r[...] = y
        new_r[...] = y + ef_blk

    full = lambda i: (0, 0)
    blk = lambda i: (i, 0)
    return pl.pallas_call(
        body,
        grid=(E // block,),
        in_specs=[
            pl.BlockSpec((block, D), blk),
            pl.BlockSpec((block, D), blk),
            pl.BlockSpec((block, D), blk),
            pl.BlockSpec((3 * D, D), full),
            pl.BlockSpec((1, D), full),
            pl.BlockSpec((D, D), full),
            pl.BlockSpec((1, D), full),
        ],
        out_specs=[pl.BlockSpec((block, D), blk)] * 2,
        out_shape=[jax.ShapeDtypeStruct((E, D), jnp.float32)] * 2,
    )(gs, gr, ef, W1, b1.reshape(1, D), W2, b2.reshape(1, D))


def _node_mlp(nodes, aggm, aggw, W1, b1, W2, b2, block):
    def body(n_r, am_r, aw_r, w1_r, b1_r, w2_r, b2_r, out_r):
        w1 = w1_r[...].astype(jnp.bfloat16)
        nf = n_r[...]
        am = (am_r[0] + am_r[1]).astype(jnp.bfloat16)
        aw = (aw_r[0] + aw_r[1]).astype(jnp.bfloat16)
        x = (jnp.dot(nf.astype(jnp.bfloat16), w1[0:D],
                     preferred_element_type=jnp.float32)
             + jnp.dot(am, w1[D:2 * D], preferred_element_type=jnp.float32)
             + jnp.dot(aw, w1[2 * D:3 * D], preferred_element_type=jnp.float32)
             + b1_r[...])
        h = jnp.maximum(x, 0.0).astype(jnp.bfloat16)
        out_r[...] = (jnp.dot(h, w2_r[...].astype(jnp.bfloat16),
                              preferred_element_type=jnp.float32)
                      + b2_r[...] + nf)

    full = lambda i: (0, 0)
    blk = lambda i: (i, 0)
    pblk = lambda i: (0, i, 0)
    return pl.pallas_call(
        body,
        grid=(N_NODES // block,),
        in_specs=[
            pl.BlockSpec((block, D), blk),
            pl.BlockSpec((NC, block, D), pblk),
            pl.BlockSpec((NC, block, D), pblk),
            pl.BlockSpec((3 * D, D), full),
            pl.BlockSpec((1, D), full),
            pl.BlockSpec((D, D), full),
            pl.BlockSpec((1, D), full),
        ],
        out_specs=pl.BlockSpec((block, D), blk),
        out_shape=jax.ShapeDtypeStruct((N_NODES, D), jnp.float32),
    )(nodes, aggm, aggw, W1, b1.reshape(1, D), W2, b2.reshape(1, D))


def kernel(node_features, mesh_edge_features, world_edge_features,
           mesh_senders, mesh_receivers, world_senders, world_receivers,
           mesh_W1, mesh_b1, mesh_W2, mesh_b2,
           world_W1, world_b1, world_W2, world_b2,
           node_W1, node_b1, node_W2, node_b2):
    ms_rows, mr_rows = _gather_pairs(node_features, mesh_senders,
                                     mesh_receivers, 32, 80)
    ws_rows, wr_rows = _gather_pairs(node_features, world_senders,
                                     world_receivers, 25, 128)

    mlp_m, new_mesh = _edge_mlp(ms_rows, mr_rows, mesh_edge_features,
                                mesh_W1, mesh_b1, mesh_W2, mesh_b2, 1000)
    mlp_w, new_world = _edge_mlp(ws_rows, wr_rows, world_edge_features,
                                 world_W1, world_b1, world_W2, world_b2, 1000)

    zeros_blk = jnp.zeros((ZB, D), jnp.float32)
    aggm = _segment_sum_partials(mlp_m, mesh_receivers, zeros_blk, 32, 80)
    aggw = _segment_sum_partials(mlp_w, world_receivers, zeros_blk, 25, 128)

    new_node = _node_mlp(node_features, aggm, aggw,
                         node_W1, node_b1, node_W2, node_b2, 1000)
    return (new_node, new_mesh, new_world)
